# baseline (speedup 1.0000x reference)
import functools

import jax
import jax.numpy as jnp
from jax import lax
from jax.experimental import pallas as pl
from jax.experimental.pallas import tpu as pltpu
from jax.experimental.pallas import tpu_sc as plsc

NC = 2
NS = 16
LANES = 16
C = 64


def _eproj_body(ea_ref, wk_ref, bk_ref, out_ref):
    out_ref[...] = (
        jnp.dot(ea_ref[...], wk_ref[...], preferred_element_type=jnp.float32)
        + bk_ref[...]
    )


def _edge_proj(edge_attr, W_e, b_e, EP):
    E, DE = edge_attr.shape
    D = W_e.shape[1]
    G = 128 // DE
    ea2 = edge_attr.reshape(E // G, G * DE)
    wk = jnp.kron(jnp.eye(G, dtype=W_e.dtype), W_e)
    bk = jnp.tile(b_e, G).reshape(1, G * D)
    BR = 320
    nreal = E // G // BR
    assert (E // G) % BR == 0 and (EP // G) % BR == 0
    return pl.pallas_call(
        _eproj_body,
        grid=(EP // G // BR,),
        in_specs=[
            pl.BlockSpec((BR, G * DE),
                         lambda i: (jnp.minimum(i, nreal - 1), 0)),
            pl.BlockSpec((G * DE, G * D), lambda i: (0, 0)),
            pl.BlockSpec((1, G * D), lambda i: (0, 0)),
        ],
        out_specs=pl.BlockSpec((BR, G * D), lambda i: (i, 0)),
        out_shape=jax.ShapeDtypeStruct((EP // G, G * D), jnp.float32),
    )(ea2, wk, bk)


def _make_sc_aggregate(NP, EP, D, NCH0):
    NCHT = EP // (NS * C)
    NCH1 = NCHT - NCH0
    NCHM = max(NCH0, NCH1)
    RPT = NP // NS
    assert NCH0 % 16 == 0 and NCH1 % 16 == 0
    assert RPT % C == 0 and D % LANES == 0
    mesh = plsc.VectorSubcoreMesh(core_axis_name="c", subcore_axis_name="s")

    @functools.partial(
        pl.kernel,
        out_type=jax.ShapeDtypeStruct((NC, NP, D), jnp.float32),
        mesh=mesh,
        scratch_types=[
            pltpu.VMEM((NCHM * C // 128, 128), jnp.int32),
            pltpu.VMEM((C,), jnp.int32),
            pltpu.VMEM((C,), jnp.int32),
            pltpu.VMEM((C,), jnp.int32),
            pltpu.VMEM((C,), jnp.int32),
            pltpu.VMEM((2, C, D), jnp.float32),
            pltpu.VMEM((2, C // 8, 8 * D), jnp.float32),
            pltpu.VMEM_SHARED((NP, D), jnp.float32),
            pltpu.SemaphoreType.DMA,
            pltpu.SemaphoreType.DMA,
            pltpu.SemaphoreType.DMA,
            pltpu.SemaphoreType.DMA,
        ],
    )
    def sc_aggregate(x_hbm, e_hbm, combo_hbm, out_hbm,
                     combo, sstage0, sstage1, dstage0, dstage1, xbuf, ebuf,
                     aggr, semx0, semx1, seme0, seme1):
        cid = lax.axis_index("c")
        sid = lax.axis_index("s")
        semx = (semx0, semx1)
        seme = (seme0, seme1)
        sstage = (sstage0, sstage1)
        dstage = (dstage0, dstage1)
        CROWS0 = NCH0 * C // 128
        CROWS1 = NCH1 * C // 128
        cbase = jnp.where(cid == 0, sid * NCH0, NS * NCH0 + sid * NCH1)
        nch = jnp.where(cid == 0, NCH0, NCH1)

        @pl.when(cid == 0)
        def _ld0():
            pltpu.sync_copy(combo_hbm.at[pl.ds(sid * CROWS0, CROWS0)],
                            combo.at[pl.ds(0, CROWS0)])

        @pl.when(cid == 1)
        def _ld1():
            pltpu.sync_copy(
                combo_hbm.at[pl.ds(NS * CROWS0 + sid * CROWS1, CROWS1)],
                combo.at[pl.ds(0, CROWS1)])

        zero = jnp.zeros((LANES,), jnp.float32)

        def zrow(i, _):
            for j in range(D // LANES):
                xbuf[0, i, pl.ds(j * LANES, LANES)] = zero
            return 0

        lax.fori_loop(0, C, zrow, 0)
        for k in range(RPT // C):
            pltpu.sync_copy(xbuf.at[0], aggr.at[pl.ds(sid * RPT + k * C, C)])
        plsc.subcore_barrier()

        def unpack(row, half, b):
            for j in range(C // LANES):
                sl = pl.ds(j * LANES, LANES)
                cv = combo[row, pl.ds(half * C + j * LANES, LANES)]
                sstage[b][sl] = cv & 0xFFFF
                dstage[b][sl] = lax.shift_right_logical(cv, 16)

        def issue(i, b):
            pltpu.async_copy(x_hbm.at[sstage[b]], xbuf.at[b], semx[b])
            pltpu.async_copy(e_hbm.at[pl.ds((cbase + i) * (C // 8), C // 8)],
                             ebuf.at[b], seme[b])

        unpack(0, 0, 0)
        issue(0, 0)

        def pair(g, _):
            for b in range(2):
                i = g * 2 + b

                @pl.when(i + 1 < nch)
                def _prefetch():
                    if b == 0:
                        unpack(g, 1, 1)
                    else:
                        unpack(g + 1, 0, 0)
                    issue(i + 1, b ^ 1)

                pltpu.make_async_copy(x_hbm.at[sstage[b]],
                                      xbuf.at[b], semx[b]).wait()
                pltpu.make_async_copy(e_hbm.at[pl.ds(0, C // 8)],
                                      ebuf.at[b], seme[b]).wait()
                xb = xbuf.at[b]
                eb = ebuf.at[b]

                def row(q, _):
                    for k in range(8):
                        r = q * 8 + k
                        for j in range(D // LANES):
                            sl = pl.ds(j * LANES, LANES)
                            xb[r, sl] = jnp.maximum(
                                xb[r, sl] + eb[q, pl.ds(k * D + j * LANES,
                                                        LANES)], 0.0)
                    return 0

                lax.fori_loop(0, C // 8, row, 0)
                pltpu.sync_copy(xb, aggr.at[dstage[b]], add=True)
            return 0

        lax.fori_loop(0, nch // 2, pair, 0)
        plsc.subcore_barrier()
        pltpu.sync_copy(
            aggr.at[pl.ds(sid * RPT, RPT)],
            out_hbm.at[cid, pl.ds(sid * RPT, RPT)],
        )

    return sc_aggregate


def _make_node_body(N):
    def _node_body(x_ref, p_ref, w1_ref, b1_ref, w2_ref, b2_ref, g_ref,
                   bt_ref, out_ref):
        h = x_ref[...] + p_ref[0, :N] + p_ref[1, :N]
        h = jnp.maximum(
            jnp.dot(h, w1_ref[...], preferred_element_type=jnp.float32)
            + b1_ref[...], 0.0)
        h = (jnp.dot(h, w2_ref[...], preferred_element_type=jnp.float32)
             + b2_ref[...])
        mean = jnp.mean(h, axis=0, keepdims=True)
        var = jnp.mean((h - mean) ** 2, axis=0, keepdims=True)
        h = g_ref[...] * (h - mean) * lax.rsqrt(var + 1e-5) + bt_ref[...]
        out_ref[...] = jnp.maximum(h, 0.0)

    return _node_body


def _node_mlp(x, partials, W1, b1, W2, b2, gamma, beta):
    N, D = x.shape
    return pl.pallas_call(
        _make_node_body(N),
        out_shape=jax.ShapeDtypeStruct((N, D), jnp.float32),
    )(x, partials, W1, b1.reshape(1, D), W2, b2.reshape(1, D),
      gamma.reshape(1, D), beta.reshape(1, D))


def kernel(x, edge_index, edge_attr, W_e, b_e, W1, b1, W2, b2, gamma, beta):
    N, D = x.shape
    E, DE = edge_attr.shape
    NP = ((N + NS * C - 1) // (NS * C)) * (NS * C)
    EPW = NC * NS * C * 8
    EP = ((E + EPW - 1) // EPW) * EPW
    src = edge_index[0].astype(jnp.int32)
    dst = edge_index[1].astype(jnp.int32)
    if EP != E:
        pad = EP - E
        src = jnp.concatenate([src, jnp.zeros((pad,), jnp.int32)])
        deadrows = N + jax.lax.rem(jnp.arange(pad, dtype=jnp.int32), NP - N)
        dst = jnp.concatenate([dst, deadrows])
    combo2d = ((dst << 16) | src).reshape(EP // 128, 128)
    e = _edge_proj(edge_attr, W_e, b_e, EP)
    NCH0 = 192
    partials = _make_sc_aggregate(NP, EP, D, NCH0)(x, e, combo2d)
    return _node_mlp(x, partials, W1, b1, W2, b2, gamma, beta)


# speedup vs baseline: 1.1125x; 1.1125x over previous
import functools

import jax
import jax.numpy as jnp
from jax import lax
from jax.experimental import pallas as pl
from jax.experimental.pallas import tpu as pltpu
from jax.experimental.pallas import tpu_sc as plsc

NC = 2
NS = 16
LANES = 16
C = 64


def _eproj_body(ea_ref, we_ref, be_ref, out_ref):
    out_ref[...] = (
        jnp.dot(ea_ref[...], we_ref[...], preferred_element_type=jnp.float32)
        + be_ref[...]
    )


def _edge_proj(edge_attr, W_e, b_e, EP):
    E, DE = edge_attr.shape
    D = W_e.shape[1]
    BE = 2560
    nreal = E // BE
    assert E % BE == 0 and EP % BE == 0
    return pl.pallas_call(
        _eproj_body,
        grid=(EP // BE,),
        in_specs=[
            pl.BlockSpec((BE, DE), lambda i: (jnp.minimum(i, nreal - 1), 0)),
            pl.BlockSpec((DE, D), lambda i: (0, 0)),
            pl.BlockSpec((1, D), lambda i: (0, 0)),
        ],
        out_specs=pl.BlockSpec((BE, D), lambda i: (i, 0)),
        out_shape=jax.ShapeDtypeStruct((EP, D), jnp.float32),
    )(edge_attr, W_e, b_e.reshape(1, D))


def _make_sc_aggregate(NP, EP, D, NCH0):
    NCHT = EP // (NS * C)
    NCH1 = NCHT - NCH0
    NCHM = max(NCH0, NCH1)
    RPT = NP // NS
    assert NCH0 % 16 == 0 and NCH1 % 16 == 0
    assert RPT % C == 0 and D % LANES == 0
    mesh = plsc.VectorSubcoreMesh(core_axis_name="c", subcore_axis_name="s")

    @functools.partial(
        pl.kernel,
        out_type=jax.ShapeDtypeStruct((NC, NP, D), jnp.float32),
        mesh=mesh,
        scratch_types=[
            pltpu.VMEM((NCHM * C // 128, 128), jnp.int32),
            pltpu.VMEM((C,), jnp.int32),
            pltpu.VMEM((C,), jnp.int32),
            pltpu.VMEM((C,), jnp.int32),
            pltpu.VMEM((C,), jnp.int32),
            pltpu.VMEM((2, C, D), jnp.float32),
            pltpu.VMEM((2, C, D), jnp.float32),
            pltpu.VMEM_SHARED((NP, D), jnp.float32),
            pltpu.SemaphoreType.DMA,
            pltpu.SemaphoreType.DMA,
            pltpu.SemaphoreType.DMA,
            pltpu.SemaphoreType.DMA,
        ],
    )
    def sc_aggregate(x_hbm, e_hbm, combo_hbm, out_hbm,
                     combo, sstage0, sstage1, dstage0, dstage1, xbuf, ebuf,
                     aggr, semx0, semx1, seme0, seme1):
        cid = lax.axis_index("c")
        sid = lax.axis_index("s")
        semx = (semx0, semx1)
        seme = (seme0, seme1)
        sstage = (sstage0, sstage1)
        dstage = (dstage0, dstage1)
        CROWS0 = NCH0 * C // 128
        CROWS1 = NCH1 * C // 128
        cbase = jnp.where(cid == 0, sid * NCH0, NS * NCH0 + sid * NCH1)
        nch = jnp.where(cid == 0, NCH0, NCH1)

        @pl.when(cid == 0)
        def _ld0():
            pltpu.sync_copy(combo_hbm.at[pl.ds(sid * CROWS0, CROWS0)],
                            combo.at[pl.ds(0, CROWS0)])

        @pl.when(cid == 1)
        def _ld1():
            pltpu.sync_copy(
                combo_hbm.at[pl.ds(NS * CROWS0 + sid * CROWS1, CROWS1)],
                combo.at[pl.ds(0, CROWS1)])

        zero = jnp.zeros((LANES,), jnp.float32)

        def zrow(i, _):
            for j in range(D // LANES):
                xbuf[0, i, pl.ds(j * LANES, LANES)] = zero
            return 0

        lax.fori_loop(0, C, zrow, 0)
        for k in range(RPT // C):
            pltpu.sync_copy(xbuf.at[0], aggr.at[pl.ds(sid * RPT + k * C, C)])
        plsc.subcore_barrier()

        def unpack(row, half, b):
            for j in range(C // LANES):
                sl = pl.ds(j * LANES, LANES)
                cv = combo[row, pl.ds(half * C + j * LANES, LANES)]
                sstage[b][sl] = cv & 0xFFFF
                dstage[b][sl] = lax.shift_right_logical(cv, 16)

        def issue(i, b):
            pltpu.async_copy(x_hbm.at[sstage[b]], xbuf.at[b], semx[b])
            pltpu.async_copy(e_hbm.at[pl.ds((cbase + i) * C, C)],
                             ebuf.at[b], seme[b])

        unpack(0, 0, 0)
        issue(0, 0)

        def pair(g, _):
            for b in range(2):
                i = g * 2 + b

                @pl.when(i + 1 < nch)
                def _prefetch():
                    if b == 0:
                        unpack(g, 1, 1)
                    else:
                        unpack(g + 1, 0, 0)
                    issue(i + 1, b ^ 1)

                pltpu.make_async_copy(x_hbm.at[sstage[b]],
                                      xbuf.at[b], semx[b]).wait()
                pltpu.make_async_copy(e_hbm.at[pl.ds(0, C)],
                                      ebuf.at[b], seme[b]).wait()
                xb = xbuf.at[b]
                eb = ebuf.at[b]

                def row(r, _):
                    for j in range(D // LANES):
                        sl = pl.ds(j * LANES, LANES)
                        eb[r, sl] = jnp.maximum(xb[r, sl] + eb[r, sl], 0.0)
                    return 0

                lax.fori_loop(0, C, row, 0)
                pltpu.sync_copy(eb, aggr.at[dstage[b]], add=True)
            return 0

        lax.fori_loop(0, nch // 2, pair, 0)
        plsc.subcore_barrier()
        pltpu.sync_copy(
            aggr.at[pl.ds(sid * RPT, RPT)],
            out_hbm.at[cid, pl.ds(sid * RPT, RPT)],
        )

    return sc_aggregate


def _make_node_body(N):
    def _node_body(x_ref, p_ref, w1_ref, b1_ref, w2_ref, b2_ref, g_ref,
                   bt_ref, out_ref):
        h = x_ref[...] + p_ref[0, :N] + p_ref[1, :N]
        h = jnp.maximum(
            jnp.dot(h, w1_ref[...], preferred_element_type=jnp.float32)
            + b1_ref[...], 0.0)
        h = (jnp.dot(h, w2_ref[...], preferred_element_type=jnp.float32)
             + b2_ref[...])
        mean = jnp.mean(h, axis=0, keepdims=True)
        var = jnp.mean((h - mean) ** 2, axis=0, keepdims=True)
        h = g_ref[...] * (h - mean) * lax.rsqrt(var + 1e-5) + bt_ref[...]
        out_ref[...] = jnp.maximum(h, 0.0)

    return _node_body


def _node_mlp(x, partials, W1, b1, W2, b2, gamma, beta):
    N, D = x.shape
    return pl.pallas_call(
        _make_node_body(N),
        out_shape=jax.ShapeDtypeStruct((N, D), jnp.float32),
    )(x, partials, W1, b1.reshape(1, D), W2, b2.reshape(1, D),
      gamma.reshape(1, D), beta.reshape(1, D))


def kernel(x, edge_index, edge_attr, W_e, b_e, W1, b1, W2, b2, gamma, beta):
    N, D = x.shape
    E, DE = edge_attr.shape
    NP = ((N + NS * C - 1) // (NS * C)) * (NS * C)
    EPW = NC * NS * C * 8
    EP = ((E + EPW - 1) // EPW) * EPW
    src = edge_index[0].astype(jnp.int32)
    dst = edge_index[1].astype(jnp.int32)
    if EP != E:
        pad = EP - E
        src = jnp.concatenate([src, jnp.zeros((pad,), jnp.int32)])
        deadrows = N + jax.lax.rem(jnp.arange(pad, dtype=jnp.int32), NP - N)
        dst = jnp.concatenate([dst, deadrows])
    combo2d = ((dst << 16) | src).reshape(EP // 128, 128)
    e = _edge_proj(edge_attr, W_e, b_e, EP)
    NCH0 = 192
    partials = _make_sc_aggregate(NP, EP, D, NCH0)(x, e, combo2d)
    return _node_mlp(x, partials, W1, b1, W2, b2, gamma, beta)


# speedup vs baseline: 1.1585x; 1.0414x over previous
import functools

import jax
import jax.numpy as jnp
from jax import lax
from jax.experimental import pallas as pl
from jax.experimental.pallas import tpu as pltpu
from jax.experimental.pallas import tpu_sc as plsc

NC = 2
NS = 16
LANES = 16
C = 64


def _eproj_body(ea_ref, we_ref, be_ref, out_ref):
    out_ref[...] = (
        jnp.dot(ea_ref[...], we_ref[...], preferred_element_type=jnp.float32)
        + be_ref[...]
    )


def _edge_proj(edge_attr, W_e, b_e, EP):
    E, DE = edge_attr.shape
    D = W_e.shape[1]
    BE = 2560
    nreal = E // BE
    assert E % BE == 0 and EP % BE == 0
    return pl.pallas_call(
        _eproj_body,
        grid=(EP // BE,),
        in_specs=[
            pl.BlockSpec((BE, DE), lambda i: (jnp.minimum(i, nreal - 1), 0)),
            pl.BlockSpec((DE, D), lambda i: (0, 0)),
            pl.BlockSpec((1, D), lambda i: (0, 0)),
        ],
        out_specs=pl.BlockSpec((BE, D), lambda i: (i, 0)),
        out_shape=jax.ShapeDtypeStruct((EP, D), jnp.float32),
    )(edge_attr, W_e, b_e.reshape(1, D))


def _make_sc_aggregate(NP, EP, D, NCH0):
    NCHT = EP // (NS * C)
    NCH1 = NCHT - NCH0
    NCHM = max(NCH0, NCH1)
    RPT = NP // NS
    assert NCH0 % 16 == 0 and NCH1 % 16 == 0
    assert RPT % C == 0 and D % LANES == 0
    mesh = plsc.VectorSubcoreMesh(core_axis_name="c", subcore_axis_name="s")

    @functools.partial(
        pl.kernel,
        out_type=jax.ShapeDtypeStruct((NC, NP, D), jnp.float32),
        mesh=mesh,
        scratch_types=[
            pltpu.VMEM((NCHM * C // 128, 128), jnp.int32),
            pltpu.VMEM((C,), jnp.int32),
            pltpu.VMEM((C,), jnp.int32),
            pltpu.VMEM((C,), jnp.int32),
            pltpu.VMEM((C,), jnp.int32),
            pltpu.VMEM((2, C, D), jnp.float32),
            pltpu.VMEM((2, C, D), jnp.float32),
            pltpu.VMEM_SHARED((NP, D), jnp.float32),
            pltpu.SemaphoreType.DMA,
            pltpu.SemaphoreType.DMA,
            pltpu.SemaphoreType.DMA,
            pltpu.SemaphoreType.DMA,
        ],
    )
    def sc_aggregate(x_hbm, e_hbm, combo_hbm, out_hbm,
                     combo, sstage0, sstage1, dstage0, dstage1, xbuf, ebuf,
                     aggr, semx0, semx1, seme0, seme1):
        cid = lax.axis_index("c")
        sid = lax.axis_index("s")
        semx = (semx0, semx1)
        seme = (seme0, seme1)
        sstage = (sstage0, sstage1)
        dstage = (dstage0, dstage1)
        CROWS0 = NCH0 * C // 128
        CROWS1 = NCH1 * C // 128
        cbase = jnp.where(cid == 0, sid * NCH0, NS * NCH0 + sid * NCH1)
        nch = jnp.where(cid == 0, NCH0, NCH1)

        @pl.when(cid == 0)
        def _ld0():
            pltpu.sync_copy(combo_hbm.at[pl.ds(sid * CROWS0, CROWS0)],
                            combo.at[pl.ds(0, CROWS0)])

        @pl.when(cid == 1)
        def _ld1():
            pltpu.sync_copy(
                combo_hbm.at[pl.ds(NS * CROWS0 + sid * CROWS1, CROWS1)],
                combo.at[pl.ds(0, CROWS1)])

        zero = jnp.zeros((LANES,), jnp.float32)

        def zrow(i, _):
            for j in range(D // LANES):
                xbuf[0, i, pl.ds(j * LANES, LANES)] = zero
            return 0

        lax.fori_loop(0, C, zrow, 0)
        for k in range(RPT // C):
            pltpu.sync_copy(xbuf.at[0], aggr.at[pl.ds(sid * RPT + k * C, C)])
        plsc.subcore_barrier()

        def unpack(row, half, b):
            for j in range(C // LANES):
                sl = pl.ds(j * LANES, LANES)
                cv = combo[row, pl.ds(half * C + j * LANES, LANES)]
                sstage[b][sl] = cv & 0xFFFF
                dstage[b][sl] = lax.shift_right_logical(cv, 16)

        def issue(i, b):
            pltpu.async_copy(x_hbm.at[sstage[b]], xbuf.at[b], semx[b])
            pltpu.async_copy(e_hbm.at[pl.ds((cbase + i) * C, C)],
                             ebuf.at[b], seme[b])

        unpack(0, 0, 0)
        issue(0, 0)

        def pair(g, _):
            for b in range(2):
                i = g * 2 + b

                @pl.when(i + 1 < nch)
                def _prefetch():
                    if b == 0:
                        unpack(g, 1, 1)
                    else:
                        unpack(g + 1, 0, 0)
                    issue(i + 1, b ^ 1)

                pltpu.make_async_copy(x_hbm.at[sstage[b]],
                                      xbuf.at[b], semx[b]).wait()
                pltpu.make_async_copy(e_hbm.at[pl.ds(0, C)],
                                      ebuf.at[b], seme[b]).wait()
                xb = xbuf.at[b]
                eb = ebuf.at[b]

                def row(r, _):
                    for j in range(D // LANES):
                        sl = pl.ds(j * LANES, LANES)
                        eb[r, sl] = jnp.maximum(xb[r, sl] + eb[r, sl], 0.0)
                    return 0

                lax.fori_loop(0, C, row, 0)
                pltpu.sync_copy(eb, aggr.at[dstage[b]], add=True)
            return 0

        lax.fori_loop(0, nch // 2, pair, 0)
        plsc.subcore_barrier()
        pltpu.sync_copy(
            aggr.at[pl.ds(sid * RPT, RPT)],
            out_hbm.at[cid, pl.ds(sid * RPT, RPT)],
        )

    return sc_aggregate


def _make_node_body(N):
    def _node_body(x_ref, p_ref, w1_ref, b1_ref, w2_ref, b2_ref, g_ref,
                   bt_ref, out_ref):
        h = x_ref[...] + p_ref[0, :N] + p_ref[1, :N]
        h = jnp.maximum(
            jnp.dot(h, w1_ref[...], preferred_element_type=jnp.float32)
            + b1_ref[...], 0.0)
        h = (jnp.dot(h, w2_ref[...], preferred_element_type=jnp.float32)
             + b2_ref[...])
        mean = jnp.mean(h, axis=0, keepdims=True)
        var = jnp.mean((h - mean) ** 2, axis=0, keepdims=True)
        h = g_ref[...] * (h - mean) * lax.rsqrt(var + 1e-5) + bt_ref[...]
        out_ref[...] = jnp.maximum(h, 0.0)

    return _node_body


def _node_mlp(x, partials, W1, b1, W2, b2, gamma, beta):
    N, D = x.shape
    return pl.pallas_call(
        _make_node_body(N),
        out_shape=jax.ShapeDtypeStruct((N, D), jnp.float32),
    )(x, partials, W1, b1.reshape(1, D), W2, b2.reshape(1, D),
      gamma.reshape(1, D), beta.reshape(1, D))


def kernel(x, edge_index, edge_attr, W_e, b_e, W1, b1, W2, b2, gamma, beta):
    N, D = x.shape
    E, DE = edge_attr.shape
    NP = ((N + NS * C - 1) // (NS * C)) * (NS * C)
    EPW = NC * NS * C * 8
    EP = ((E + EPW - 1) // EPW) * EPW
    src = edge_index[0].astype(jnp.int32)
    dst = edge_index[1].astype(jnp.int32)
    if EP != E:
        pad = EP - E
        src = jnp.concatenate([src, jnp.zeros((pad,), jnp.int32)])
        deadrows = N + jax.lax.rem(jnp.arange(pad, dtype=jnp.int32), NP - N)
        dst = jnp.concatenate([dst, deadrows])
    combo2d = ((dst << 16) | src).reshape(EP // 128, 128)
    e = _edge_proj(edge_attr, W_e, b_e, EP)
    NCH0 = 240
    partials = _make_sc_aggregate(NP, EP, D, NCH0)(x, e, combo2d)
    return _node_mlp(x, partials, W1, b1, W2, b2, gamma, beta)
